# speedup vs baseline: 1.4800x; 1.4800x over previous
import functools

import jax
import jax.numpy as jnp
from jax import lax
from jax.experimental import pallas as pl
from jax.experimental.pallas import tpu as pltpu
from jax.experimental.pallas import tpu_sc as plsc

_NC = 2
_NS = 16
_NW = _NC * _NS


@functools.cache
def _make_lookup(B, V, D):
    b_per_w = B // _NW
    C = 8
    n_chunks = b_per_w // C
    mesh = plsc.VectorSubcoreMesh(core_axis_name="c", subcore_axis_name="s")

    @functools.partial(
        pl.kernel,
        mesh=mesh,
        out_type=jax.ShapeDtypeStruct((B, D), jnp.float32),
        scratch_types=[
            pltpu.VMEM((b_per_w,), jnp.int32),
            pltpu.VMEM((C, D), jnp.float32),
            pltpu.SemaphoreType.DMA,
        ],
    )
    def lookup(idx_hbm, table_hbm, out_hbm, idx_v, rows_v, sem):
        wid = lax.axis_index("s") * _NC + lax.axis_index("c")
        base = wid * b_per_w
        pltpu.sync_copy(idx_hbm.at[pl.ds(base, b_per_w)], idx_v)

        def body(c):
            pltpu.async_copy(
                table_hbm.at[idx_v.at[pl.ds(c * C, C)]], rows_v, sem
            ).wait()
            pltpu.sync_copy(rows_v, out_hbm.at[pl.ds(base + c * C, C)])

        pl.loop(0, n_chunks)(body)

    return lookup


def kernel(token_ids, table):
    V, D = table.shape
    idx = token_ids.reshape(-1).astype(jnp.int32)
    out = _make_lookup(idx.shape[0], V, D)(idx, table)
    return out.reshape(token_ids.shape + (D,))


# speedup vs baseline: 1.7580x; 1.1878x over previous
import functools

import jax
import jax.numpy as jnp
from jax import lax
from jax.experimental import pallas as pl
from jax.experimental.pallas import tpu as pltpu
from jax.experimental.pallas import tpu_sc as plsc

_NC = 2
_NS = 16
_NW = _NC * _NS


@functools.cache
def _make_lookup(B, V, D):
    b_per_w = B // _NW
    C = 8
    NBUF = 2
    n_chunks = b_per_w // C
    mesh = plsc.VectorSubcoreMesh(core_axis_name="c", subcore_axis_name="s")

    @functools.partial(
        pl.kernel,
        mesh=mesh,
        out_type=jax.ShapeDtypeStruct((B, D), jnp.float32),
        scratch_types=[
            pltpu.VMEM((b_per_w,), jnp.int32),
            [pltpu.VMEM((C, D), jnp.float32) for _ in range(NBUF)],
            [pltpu.SemaphoreType.DMA for _ in range(NBUF)],
            [pltpu.SemaphoreType.DMA for _ in range(NBUF)],
        ],
    )
    def lookup(idx_hbm, table_hbm, out_hbm, idx_v, bufs, gsems, wsems):
        wid = lax.axis_index("s") * _NC + lax.axis_index("c")
        base = wid * b_per_w
        pltpu.sync_copy(idx_hbm.at[pl.ds(base, b_per_w)], idx_v)

        def fire_gather(b, c):
            pltpu.async_copy(
                table_hbm.at[idx_v.at[pl.ds(c * C, C)]], bufs[b], gsems[b]
            )

        def wait_gather(b):
            pltpu.make_async_copy(out_hbm.at[pl.ds(base, C)], bufs[b],
                                  gsems[b]).wait()

        def fire_write(b, c):
            pltpu.async_copy(
                bufs[b], out_hbm.at[pl.ds(base + c * C, C)], wsems[b]
            )

        def wait_write(b):
            pltpu.make_async_copy(bufs[b], out_hbm.at[pl.ds(base, C)],
                                  wsems[b]).wait()

        for b in range(NBUF):
            fire_gather(b, b)

        def body(g):
            for b in range(NBUF):
                c = g + b
                wait_gather(b)
                fire_write(b, c)

                @pl.when(c + NBUF < n_chunks)
                def _():
                    wait_write(b)
                    fire_gather(b, c + NBUF)

        pl.loop(0, n_chunks, step=NBUF)(body)

        for b in range(NBUF):
            wait_write(b)

    return lookup


def kernel(token_ids, table):
    V, D = table.shape
    idx = token_ids.reshape(-1).astype(jnp.int32)
    out = _make_lookup(idx.shape[0], V, D)(idx, table)
    return out.reshape(token_ids.shape + (D,))


# speedup vs baseline: 1.7690x; 1.0063x over previous
import functools

import jax
import jax.numpy as jnp
from jax import lax
from jax.experimental import pallas as pl
from jax.experimental.pallas import tpu as pltpu
from jax.experimental.pallas import tpu_sc as plsc

_NC = 2
_NS = 16
_NW = _NC * _NS


@functools.cache
def _make_lookup(B, V, D):
    b_per_w = B // _NW
    C = 8
    NBUF = 3
    n_chunks = b_per_w // C
    mesh = plsc.VectorSubcoreMesh(core_axis_name="c", subcore_axis_name="s")

    @functools.partial(
        pl.kernel,
        mesh=mesh,
        out_type=jax.ShapeDtypeStruct((B, D), jnp.float32),
        scratch_types=[
            pltpu.VMEM((b_per_w,), jnp.int32),
            [pltpu.VMEM((C, D), jnp.float32) for _ in range(NBUF)],
            [pltpu.SemaphoreType.DMA for _ in range(NBUF)],
            [pltpu.SemaphoreType.DMA for _ in range(NBUF)],
        ],
    )
    def lookup(idx_hbm, table_hbm, out_hbm, idx_v, bufs, gsems, wsems):
        wid = lax.axis_index("s") * _NC + lax.axis_index("c")
        base = wid * b_per_w
        pltpu.sync_copy(idx_hbm.at[pl.ds(base, b_per_w)], idx_v)

        def fire_gather(b, c):
            pltpu.async_copy(
                table_hbm.at[idx_v.at[pl.ds(c * C, C)]], bufs[b], gsems[b]
            )

        def wait_gather(b):
            pltpu.make_async_copy(out_hbm.at[pl.ds(base, C)], bufs[b],
                                  gsems[b]).wait()

        def fire_write(b, c):
            pltpu.async_copy(
                bufs[b], out_hbm.at[pl.ds(base + c * C, C)], wsems[b]
            )

        def wait_write(b):
            pltpu.make_async_copy(bufs[b], out_hbm.at[pl.ds(base, C)],
                                  wsems[b]).wait()

        for b in range(NBUF):
            fire_gather(b, b)

        def body(g):
            for b in range(NBUF):
                c = g + b
                wait_gather(b)
                fire_write(b, c)

                @pl.when(c + NBUF < n_chunks)
                def _():
                    wait_write(b)
                    fire_gather(b, c + NBUF)

        main = n_chunks - n_chunks % NBUF
        pl.loop(0, main, step=NBUF)(body)

        for i in range(main, n_chunks):
            wait_gather(i % NBUF)
            fire_write(i % NBUF, i)

        for b in range(NBUF):
            wait_write(b)

    return lookup


def kernel(token_ids, table):
    V, D = table.shape
    idx = token_ids.reshape(-1).astype(jnp.int32)
    out = _make_lookup(idx.shape[0], V, D)(idx, table)
    return out.reshape(token_ids.shape + (D,))
